# baseline (speedup 1.0000x reference)
import functools

import jax
import jax.numpy as jnp
from jax import lax
from jax.experimental import pallas as pl
from jax.experimental.pallas import tpu as pltpu
from jax.experimental.pallas import tpu_sc as plsc

_D0 = 1048576
_NC = 2
_NS = 16
_NW = _NC * _NS
_D0_W = _D0 // _NW
_CH = 2048
_NCHUNK = _D0_W // _CH
_LANES = 16


_NIN = 3
_NOUT = 2


def _tec_body(x_hbm, out_hbm, *scratch):
    wid = lax.axis_index("s") * _NC + lax.axis_index("c")
    base = wid * _D0_W
    inbuf = scratch[0:_NIN]
    outbuf = scratch[_NIN:_NIN + _NOUT]
    isem = scratch[_NIN + _NOUT:2 * _NIN + _NOUT]
    osem = scratch[2 * _NIN + _NOUT:2 * _NIN + 2 * _NOUT]

    def load(c):
        return pltpu.async_copy(
            x_hbm.at[:, :, pl.ds(base + c * _CH, _CH)],
            inbuf[c % _NIN], isem[c % _NIN])

    def store(c):
        return pltpu.async_copy(
            outbuf[c % _NOUT], out_hbm.at[:, :, pl.ds(base + c * _CH, _CH)],
            osem[c % _NOUT])

    loads = {}
    for c in range(min(_NIN, _NCHUNK)):
        loads[c] = load(c)
    stores = {}
    for c in range(_NCHUNK):
        loads[c].wait()
        if c >= _NOUT:
            stores[c - _NOUT].wait()
        src = inbuf[c % _NIN]
        dst = outbuf[c % _NOUT]

        @plsc.parallel_loop(0, _CH // _LANES, unroll=4)
        def _vec(i):
            o = i * _LANES
            for d1 in range(3):
                for d2 in range(3):
                    v = src[d1, d2, pl.ds(o, _LANES)]
                    dst[d1, d2, pl.ds(o, _LANES)] = jnp.where(
                        v == 0.0, v, jnp.float32(2.0))

        stores[c] = store(c)
        if c + _NIN < _NCHUNK:
            loads[c + _NIN] = load(c + _NIN)
    for c in range(max(0, _NCHUNK - _NOUT), _NCHUNK):
        stores[c].wait()


@functools.partial(
    pl.kernel,
    mesh=plsc.VectorSubcoreMesh(core_axis_name="c", subcore_axis_name="s"),
    out_type=jax.ShapeDtypeStruct((3, 3, _D0), jnp.float32),
    compiler_params=pltpu.CompilerParams(use_tc_tiling_on_sc=True),
    scratch_types=(
        [pltpu.VMEM((3, 3, _CH), jnp.float32)] * (_NIN + _NOUT)
        + [pltpu.SemaphoreType.DMA] * (_NIN + _NOUT)
    ),
)
def _sc_mask_set(x_hbm, out_hbm, *scratch):
    _tec_body(x_hbm, out_hbm, *scratch)


def kernel(x):
    xt = jnp.transpose(x, (1, 2, 0))
    ot = _sc_mask_set(xt)
    return jnp.transpose(ot, (2, 0, 1))


# speedup vs baseline: 1.0930x; 1.0930x over previous
import functools

import jax
import jax.numpy as jnp
from jax import lax
from jax.experimental import pallas as pl
from jax.experimental.pallas import tpu as pltpu
from jax.experimental.pallas import tpu_sc as plsc

_D0 = 1048576
_NC = 2
_NS = 16
_NW = _NC * _NS
_D0_W = _D0 // _NW
_CH = 2048
_NCHUNK = _D0_W // _CH
_LANES = 16


_NIN = 3
_NOUT = 2


def _tec_body(x_hbm, out_hbm, *scratch):
    wid = lax.axis_index("s") * _NC + lax.axis_index("c")
    base = wid * _D0_W
    inbuf = scratch[0:_NIN]
    outbuf = scratch[_NIN:_NIN + _NOUT]
    isem = scratch[_NIN + _NOUT:2 * _NIN + _NOUT]
    osem = scratch[2 * _NIN + _NOUT:2 * _NIN + 2 * _NOUT]

    def load(c):
        return pltpu.async_copy(
            x_hbm.at[:, :, pl.ds(base + c * _CH, _CH)],
            inbuf[c % _NIN], isem[c % _NIN])

    def store(c):
        return pltpu.async_copy(
            outbuf[c % _NOUT], out_hbm.at[:, :, pl.ds(base + c * _CH, _CH)],
            osem[c % _NOUT])

    loads = {}
    for c in range(min(_NIN, _NCHUNK)):
        loads[c] = load(c)
    stores = {}
    for c in range(_NCHUNK):
        loads[c].wait()
        if c >= _NOUT:
            stores[c - _NOUT].wait()
        src = inbuf[c % _NIN]
        dst = outbuf[c % _NOUT]

        @plsc.parallel_loop(0, _CH // _LANES, unroll=1)
        def _vec(i):
            o = i * _LANES
            for d1 in range(3):
                for d2 in range(3):
                    v = src[d1, d2, pl.ds(o, _LANES)]
                    dst[d1, d2, pl.ds(o, _LANES)] = jnp.where(
                        v == 0.0, v, jnp.float32(2.0))

        stores[c] = store(c)
        if c + _NIN < _NCHUNK:
            loads[c + _NIN] = load(c + _NIN)
    for c in range(max(0, _NCHUNK - _NOUT), _NCHUNK):
        stores[c].wait()


@functools.partial(
    pl.kernel,
    mesh=plsc.VectorSubcoreMesh(core_axis_name="c", subcore_axis_name="s"),
    out_type=jax.ShapeDtypeStruct((3, 3, _D0), jnp.float32),
    compiler_params=pltpu.CompilerParams(use_tc_tiling_on_sc=True),
    scratch_types=(
        [pltpu.VMEM((3, 3, _CH), jnp.float32)] * (_NIN + _NOUT)
        + [pltpu.SemaphoreType.DMA] * (_NIN + _NOUT)
    ),
)
def _sc_mask_set(x_hbm, out_hbm, *scratch):
    _tec_body(x_hbm, out_hbm, *scratch)


def kernel(x):
    xt = jnp.transpose(x, (1, 2, 0))
    ot = _sc_mask_set(xt)
    return jnp.transpose(ot, (2, 0, 1))
